# Initial kernel scaffold; baseline (speedup 1.0000x reference)
import jax
import jax.numpy as jnp
from jax.experimental import pallas as pl


def kernel(t, pe):
    raise NotImplementedError("write your pallas kernel here")


# speedup vs baseline: 1.9841x; 1.9841x over previous
import functools

import jax
import jax.numpy as jnp
from jax import lax
from jax.experimental import pallas as pl
from jax.experimental.pallas import tpu as pltpu
from jax.experimental.pallas import tpu_sc as plsc

D_MODEL = 1024
BATCH = 4 * 8192
NUM_WORKERS = 32
B_PER_W = BATCH // NUM_WORKERS
CHUNK = 32
NCHUNK = B_PER_W // CHUNK

_mesh = plsc.VectorSubcoreMesh(core_axis_name="c", subcore_axis_name="s")


@functools.partial(
    pl.kernel,
    mesh=_mesh,
    out_type=jax.ShapeDtypeStruct((BATCH, D_MODEL), jnp.float32),
    scratch_types=[
        pltpu.VMEM((NCHUNK, CHUNK), jnp.int32),
        pltpu.VMEM((CHUNK, D_MODEL), jnp.float32),
        pltpu.SemaphoreType.DMA,
    ],
)
def _gather_kernel(t_hbm, pe_hbm, out_hbm, idx_v, rows_v, sem):
    wid = lax.axis_index("s") * 2 + lax.axis_index("c")
    base = wid * B_PER_W
    pltpu.sync_copy(t_hbm.at[wid], idx_v)

    def body(c, _):
        pltpu.async_copy(pe_hbm.at[idx_v.at[c]], rows_v, sem).wait()
        pltpu.sync_copy(rows_v, out_hbm.at[pl.ds(base + c * CHUNK, CHUNK)])
        return ()

    lax.fori_loop(0, NCHUNK, body, (), unroll=False)


def kernel(t, pe):
    t_flat = t.reshape(NUM_WORKERS, NCHUNK, CHUNK)
    out = _gather_kernel(t_flat, pe)
    return out.reshape(t.shape + (D_MODEL,))


# speedup vs baseline: 2.3676x; 1.1933x over previous
import functools

import jax
import jax.numpy as jnp
from jax import lax
from jax.experimental import pallas as pl
from jax.experimental.pallas import tpu as pltpu
from jax.experimental.pallas import tpu_sc as plsc

D_MODEL = 1024
BATCH = 4 * 8192
NUM_WORKERS = 32
B_PER_W = BATCH // NUM_WORKERS
CHUNK = 32
NCHUNK = B_PER_W // CHUNK
NBUF = 2
NOUTER = NCHUNK // NBUF

_mesh = plsc.VectorSubcoreMesh(core_axis_name="c", subcore_axis_name="s")


@functools.partial(
    pl.kernel,
    mesh=_mesh,
    out_type=jax.ShapeDtypeStruct((BATCH, D_MODEL), jnp.float32),
    scratch_types=[
        pltpu.VMEM((NCHUNK, CHUNK), jnp.int32),
        pltpu.VMEM((NBUF, CHUNK, D_MODEL), jnp.float32),
        [pltpu.SemaphoreType.DMA] * NBUF,
        [pltpu.SemaphoreType.DMA] * NBUF,
    ],
)
def _gather_kernel(t_hbm, pe_hbm, out_hbm, idx_v, rows_v, gsems, ssems):
    wid = lax.axis_index("s") * 2 + lax.axis_index("c")
    base = wid * B_PER_W
    pltpu.sync_copy(t_hbm.at[wid], idx_v)

    for b in range(NBUF):
        pltpu.async_copy(pe_hbm.at[idx_v.at[b]], rows_v.at[b], gsems[b])

    def outer(i, _):
        for b in range(NBUF):
            c = i * NBUF + b
            pltpu.make_async_copy(
                pe_hbm.at[idx_v.at[b]], rows_v.at[b], gsems[b]).wait()
            pltpu.async_copy(
                rows_v.at[b], out_hbm.at[pl.ds(base + c * CHUNK, CHUNK)],
                ssems[b])

            @pl.when(i < NOUTER - 1)
            def _():
                pltpu.make_async_copy(
                    rows_v.at[b], out_hbm.at[pl.ds(base, CHUNK)],
                    ssems[b]).wait()
                pltpu.async_copy(
                    pe_hbm.at[idx_v.at[c + NBUF]], rows_v.at[b], gsems[b])

        return ()

    lax.fori_loop(0, NOUTER, outer, (), unroll=False)

    for b in range(NBUF):
        pltpu.make_async_copy(
            rows_v.at[b], out_hbm.at[pl.ds(base, CHUNK)], ssems[b]).wait()


def kernel(t, pe):
    t_flat = t.reshape(NUM_WORKERS, NCHUNK, CHUNK)
    out = _gather_kernel(t_flat, pe)
    return out.reshape(t.shape + (D_MODEL,))


# speedup vs baseline: 2.3801x; 1.0053x over previous
import functools

import jax
import jax.numpy as jnp
from jax import lax
from jax.experimental import pallas as pl
from jax.experimental.pallas import tpu as pltpu
from jax.experimental.pallas import tpu_sc as plsc

D_MODEL = 1024
BATCH = 4 * 8192
NUM_WORKERS = 32
B_PER_W = BATCH // NUM_WORKERS
CHUNK = 16
NCHUNK = B_PER_W // CHUNK
NBUF = 4
NOUTER = NCHUNK // NBUF

_mesh = plsc.VectorSubcoreMesh(core_axis_name="c", subcore_axis_name="s")


@functools.partial(
    pl.kernel,
    mesh=_mesh,
    out_type=jax.ShapeDtypeStruct((BATCH, D_MODEL), jnp.float32),
    scratch_types=[
        pltpu.VMEM((NCHUNK, CHUNK), jnp.int32),
        pltpu.VMEM((NBUF, CHUNK, D_MODEL), jnp.float32),
        [pltpu.SemaphoreType.DMA] * NBUF,
        [pltpu.SemaphoreType.DMA] * NBUF,
    ],
)
def _gather_kernel(t_hbm, pe_hbm, out_hbm, idx_v, rows_v, gsems, ssems):
    wid = lax.axis_index("s") * 2 + lax.axis_index("c")
    base = wid * B_PER_W
    pltpu.sync_copy(t_hbm.at[wid], idx_v)

    for b in range(NBUF):
        pltpu.async_copy(pe_hbm.at[idx_v.at[b]], rows_v.at[b], gsems[b])

    def outer(i, _):
        for b in range(NBUF):
            c = i * NBUF + b
            pltpu.make_async_copy(
                pe_hbm.at[idx_v.at[b]], rows_v.at[b], gsems[b]).wait()
            pltpu.async_copy(
                rows_v.at[b], out_hbm.at[pl.ds(base + c * CHUNK, CHUNK)],
                ssems[b])

            @pl.when(i < NOUTER - 1)
            def _():
                pltpu.make_async_copy(
                    rows_v.at[b], out_hbm.at[pl.ds(base, CHUNK)],
                    ssems[b]).wait()
                pltpu.async_copy(
                    pe_hbm.at[idx_v.at[c + NBUF]], rows_v.at[b], gsems[b])

        return ()

    lax.fori_loop(0, NOUTER, outer, (), unroll=False)

    for b in range(NBUF):
        pltpu.make_async_copy(
            rows_v.at[b], out_hbm.at[pl.ds(base, CHUNK)], ssems[b]).wait()


def kernel(t, pe):
    t_flat = t.reshape(NUM_WORKERS, NCHUNK, CHUNK)
    out = _gather_kernel(t_flat, pe)
    return out.reshape(t.shape + (D_MODEL,))
